# baseline (speedup 1.0000x reference)
import functools

import jax
import jax.numpy as jnp
from jax import lax
from jax.experimental import pallas as pl
from jax.experimental.pallas import tpu as pltpu
from jax.experimental.pallas import tpu_sc as plsc

N = 10000
E = 320000
H = 128
EPS = 1e-5

NC = 2
NS = 16
NW = NC * NS
NP = 10240
RPS = NP // NS
DW = 128
E2 = NW * 10240
EPW = E2 // NW
C = 80
K = EPW // C
KG = 32
G = K // KG
ZT = RPS // C

_mesh = plsc.VectorSubcoreMesh(core_axis_name="c", subcore_axis_name="s")


def _zero_stripe(buf, acc, si):
    zv = jnp.zeros((16,), jnp.float32)
    width = buf.shape[1]

    def zrow(r, c2):
        for q in range(width // 16):
            buf[r, pl.ds(q * 16, 16)] = zv
        return c2

    lax.fori_loop(0, C, zrow, 0)
    for t in range(ZT):
        pltpu.sync_copy(buf, acc.at[pl.ds(si * RPS + t * C, C)])

@functools.partial(
    pl.kernel,
    out_type=jax.ShapeDtypeStruct((NC, NP, DW), jnp.float32),
    mesh=_mesh,
    scratch_types=[
        pltpu.VMEM((KG, C), jnp.int32),
        pltpu.VMEM((C, DW), jnp.float32),
        pltpu.VMEM_SHARED((NP, DW), jnp.float32),
    ],
)
def _sc_degree(dst_hbm, ones_hbm, out_hbm, dst_v, ones_v, acc):
    ci = lax.axis_index("c")
    si = lax.axis_index("s")
    wid = si * NC + ci
    _zero_stripe(ones_v, acc, si)
    pltpu.sync_copy(ones_hbm, ones_v)
    plsc.subcore_barrier()

    def group(gr, carry):
        pltpu.sync_copy(dst_hbm.at[wid].at[gr], dst_v)

        def body(j, c2):
            pltpu.sync_copy(ones_v, acc.at[dst_v.at[j]], add=True)
            return c2

        lax.fori_loop(0, KG, body, 0)
        return carry

    lax.fori_loop(0, G, group, 0)
    plsc.subcore_barrier()
    pltpu.sync_copy(acc.at[pl.ds(si * RPS, RPS)],
                    out_hbm.at[ci].at[pl.ds(si * RPS, RPS)])


@functools.partial(
    pl.kernel,
    out_type=jax.ShapeDtypeStruct((NC, NP, H), jnp.float32),
    mesh=_mesh,
    scratch_types=[
        pltpu.VMEM((KG, C), jnp.int32),
        pltpu.VMEM((KG, C), jnp.int32),
        pltpu.VMEM((C, H), jnp.float32),
        pltpu.VMEM((C, H), jnp.float32),
        pltpu.VMEM_SHARED((NP, H), jnp.float32),
        pltpu.SemaphoreType.DMA,
        pltpu.SemaphoreType.DMA,
        pltpu.SemaphoreType.DMA,
        pltpu.SemaphoreType.DMA,
    ],
)
def _sc_segsum(g_hbm, src_hbm, dst_hbm, z_hbm, out_hbm,
               src_v, dst_v, buf0, buf1, acc, gs0, gs1, ss0, ss1):
    ci = lax.axis_index("c")
    si = lax.axis_index("s")
    wid = si * NC + ci
    pltpu.sync_copy(z_hbm.at[pl.ds(si * RPS, RPS)], acc.at[pl.ds(si * RPS, RPS)])
    plsc.subcore_barrier()

    def group(gr, carry):
        pltpu.sync_copy(src_hbm.at[wid].at[gr], src_v)
        pltpu.sync_copy(dst_hbm.at[wid].at[gr], dst_v)

        pltpu.async_copy(g_hbm.at[src_v.at[0]], buf0, gs0)

        def pair(m, c2):
            j0 = 2 * m
            j1 = j0 + 1
            pltpu.make_async_copy(g_hbm.at[src_v.at[j0]], buf0, gs0).wait()
            pltpu.async_copy(g_hbm.at[src_v.at[j1]], buf1, gs1)
            pltpu.sync_copy(buf0, acc.at[dst_v.at[j0]], add=True)
            pltpu.make_async_copy(g_hbm.at[src_v.at[j1]], buf1, gs1).wait()

            @pl.when(j1 + 1 < KG)
            def _():
                pltpu.async_copy(g_hbm.at[src_v.at[j1 + 1]], buf0, gs0)

            pltpu.sync_copy(buf1, acc.at[dst_v.at[j1]], add=True)
            return c2

        lax.fori_loop(0, KG // 2, pair, 0)
        return carry

    lax.fori_loop(0, G, group, 0)
    plsc.subcore_barrier()
    pltpu.sync_copy(acc.at[pl.ds(si * RPS, RPS)],
                    out_hbm.at[ci].at[pl.ds(si * RPS, RPS)])


def _tc0_body(x_ref, w_ref, degp_ref, g_ref, dinv_ref):
    deg = 1.0 + degp_ref[0, :N, 0:1] + degp_ref[1, :N, 0:1]
    dinv = lax.rsqrt(deg)
    dinv_ref[...] = dinv
    z = jnp.dot(x_ref[...], w_ref[...], preferred_element_type=jnp.float32)
    g_ref[...] = z * dinv


def _tc_mid_body(p_ref, g_ref, dinv_ref, b_ref, lnw_ref, lnb_ref, wn_ref,
                 h_ref, gn_ref):
    dinv = dinv_ref[...]
    u = (p_ref[0, :N] + p_ref[1, :N] + g_ref[...]) * dinv + b_ref[...]
    mean = jnp.mean(u)
    var = jnp.mean((u - mean) ** 2)
    t = (u - mean) * lax.rsqrt(var + EPS) * lnw_ref[...] + lnb_ref[...]
    h = jnp.maximum(t, 0.0)
    h_ref[...] = h
    z = jnp.dot(h, wn_ref[...], preferred_element_type=jnp.float32)
    gn_ref[...] = z * dinv


def _tc_last_body(p_ref, g_ref, dinv_ref, b_ref, lnw_ref, lnb_ref, h_ref):
    u = (p_ref[0, :N] + p_ref[1, :N] + g_ref[...]) * dinv_ref[...] + b_ref[...]
    mean = jnp.mean(u)
    var = jnp.mean((u - mean) ** 2)
    t = (u - mean) * lax.rsqrt(var + EPS) * lnw_ref[...] + lnb_ref[...]
    h_ref[...] = jnp.maximum(t, 0.0)


def _tc_head_body(h1_ref, h2_ref, h3_ref, mask_ref, wjk_ref, bjk_ref,
                  wfc1_ref, bfc1_ref, wfc2_ref, bfc2_ref, out_ref):
    f = (jnp.dot(h1_ref[...], wjk_ref[0:H], preferred_element_type=jnp.float32)
         + jnp.dot(h2_ref[...], wjk_ref[H:2 * H], preferred_element_type=jnp.float32)
         + jnp.dot(h3_ref[...], wjk_ref[2 * H:3 * H], preferred_element_type=jnp.float32)
         + bjk_ref[...])
    t = jnp.tanh(jnp.dot(f, wfc1_ref[...], preferred_element_type=jnp.float32)
                 + bfc1_ref[...])
    s = jnp.dot(t, wfc2_ref[...], preferred_element_type=jnp.float32) + bfc2_ref[...]
    s = s + mask_ref[...] * -1000000000.0
    m = jnp.max(s)
    e = jnp.exp(s - m)
    out_ref[...] = e / jnp.sum(e)


def _tc_call(body, out_shapes, *args):
    return pl.pallas_call(
        body,
        out_shape=out_shapes,
    )(*args)


def kernel(x, edge_index, mask, W1, b1, W2, b2, W3, b3, ln_w1, ln_b1,
           ln_w2, ln_b2, ln_w3, ln_b3, Wjk, bjk, Wfc1, bfc1, Wfc2, bfc2):
    ppw = EPW - E // NW
    pad_src = jnp.zeros((NW, ppw), edge_index.dtype)
    pad_dst = jnp.broadcast_to(jnp.arange(N, N + ppw, dtype=edge_index.dtype),
                               (NW, ppw))
    src = jnp.concatenate([edge_index[0].reshape(NW, E // NW), pad_src],
                          axis=1).reshape(NW, G, KG, C)
    dst = jnp.concatenate([edge_index[1].reshape(NW, E // NW), pad_dst],
                          axis=1).reshape(NW, G, KG, C)
    ones_d = jnp.ones((C, DW), jnp.float32)
    zeros_h = jnp.zeros((NP, H), jnp.float32)

    degp = _sc_degree(dst, ones_d)

    f32 = jnp.float32
    g1, dinv = _tc_call(
        _tc0_body,
        (jax.ShapeDtypeStruct((N, H), f32), jax.ShapeDtypeStruct((N, 1), f32)),
        x, W1, degp)

    nh = jax.ShapeDtypeStruct((N, H), f32)
    p1 = _sc_segsum(g1, src, dst, zeros_h)
    h1, g2 = _tc_call(
        _tc_mid_body, (nh, nh),
        p1, g1, dinv, b1.reshape(1, H), ln_w1.reshape(1, H),
        ln_b1.reshape(1, H), W2)

    p2 = _sc_segsum(g2, src, dst, zeros_h)
    h2, g3 = _tc_call(
        _tc_mid_body, (nh, nh),
        p2, g2, dinv, b2.reshape(1, H), ln_w2.reshape(1, H),
        ln_b2.reshape(1, H), W3)

    p3 = _sc_segsum(g3, src, dst, zeros_h)
    h3 = _tc_call(
        _tc_last_body, nh,
        p3, g3, dinv, b3.reshape(1, H), ln_w3.reshape(1, H),
        ln_b3.reshape(1, H))

    prob = _tc_call(
        _tc_head_body, jax.ShapeDtypeStruct((N, 1), f32),
        h1, h2, h3, mask.astype(f32), Wjk, bjk.reshape(1, H),
        Wfc1, bfc1.reshape(1, H // 2), Wfc2, bfc2.reshape(1, 1))

    return prob.reshape(N)


# speedup vs baseline: 2.2644x; 2.2644x over previous
import functools

import jax
import jax.numpy as jnp
from jax import lax
from jax.experimental import pallas as pl
from jax.experimental.pallas import tpu as pltpu
from jax.experimental.pallas import tpu_sc as plsc

N = 10000
E = 320000
H = 128
EPS = 1e-5

NC = 2
NS = 16
NW = NC * NS
NP = 10240
RPS = NP // NS
DW = 128
EPW = E // NW
C = 80
K = EPW // C
KG = 25
G = K // KG
ZT = RPS // C

_mesh = plsc.VectorSubcoreMesh(core_axis_name="c", subcore_axis_name="s")


def _zero_stripe(buf, acc, si):
    zv = jnp.zeros((16,), jnp.float32)
    width = buf.shape[1]

    def zrow(r, c2):
        for q in range(width // 16):
            buf[r, pl.ds(q * 16, 16)] = zv
        return c2

    lax.fori_loop(0, C, zrow, 0)
    for t in range(ZT):
        pltpu.sync_copy(buf, acc.at[pl.ds(si * RPS + t * C, C)])

@functools.partial(
    pl.kernel,
    out_type=jax.ShapeDtypeStruct((NC, NP, DW), jnp.float32),
    mesh=_mesh,
    scratch_types=[
        pltpu.VMEM((KG, C), jnp.int32),
        pltpu.VMEM((C, DW), jnp.float32),
        pltpu.VMEM_SHARED((NP, DW), jnp.float32),
    ],
)
def _sc_degree(dst_hbm, ones_hbm, out_hbm, dst_v, ones_v, acc):
    ci = lax.axis_index("c")
    si = lax.axis_index("s")
    wid = si * NC + ci
    _zero_stripe(ones_v, acc, si)
    pltpu.sync_copy(ones_hbm, ones_v)
    plsc.subcore_barrier()

    def group(gr, carry):
        pltpu.sync_copy(dst_hbm.at[wid].at[gr], dst_v)

        def body(j, c2):
            pltpu.sync_copy(ones_v, acc.at[dst_v.at[j]], add=True)
            return c2

        lax.fori_loop(0, KG, body, 0)
        return carry

    lax.fori_loop(0, G, group, 0)
    plsc.subcore_barrier()
    pltpu.sync_copy(acc.at[pl.ds(si * RPS, RPS)],
                    out_hbm.at[ci].at[pl.ds(si * RPS, RPS)])


@functools.partial(
    pl.kernel,
    out_type=jax.ShapeDtypeStruct((NC, NP, H), jnp.float32),
    mesh=_mesh,
    scratch_types=[
        pltpu.VMEM((KG, C), jnp.int32),
        pltpu.VMEM((KG, C), jnp.int32),
        pltpu.VMEM((C, H), jnp.float32),
        pltpu.VMEM((C, H), jnp.float32),
        pltpu.VMEM_SHARED((NP, H), jnp.float32),
        pltpu.SemaphoreType.DMA,
        pltpu.SemaphoreType.DMA,
    ],
)
def _sc_segsum(g_hbm, src_hbm, dst_hbm, z_hbm, out_hbm,
               src_v, dst_v, buf0, buf1, acc, sem0, sem1):
    ci = lax.axis_index("c")
    si = lax.axis_index("s")
    wid = si * NC + ci
    pltpu.sync_copy(z_hbm.at[pl.ds(si * RPS, RPS)], acc.at[pl.ds(si * RPS, RPS)])
    plsc.subcore_barrier()

    def group(gr, carry):
        pltpu.sync_copy(src_hbm.at[wid].at[gr], src_v)
        pltpu.sync_copy(dst_hbm.at[wid].at[gr], dst_v)

        pltpu.make_async_copy(g_hbm.at[src_v.at[0]], buf0, sem0).start()

        def body(j, c2):
            nxt = j + 1

            @pl.when(j % 2 == 0)
            def _():
                pltpu.make_async_copy(g_hbm.at[src_v.at[j]], buf0, sem0).wait()

                @pl.when(nxt < KG)
                def _():
                    pltpu.make_async_copy(g_hbm.at[src_v.at[nxt]], buf1, sem1).start()

                pltpu.sync_copy(buf0, acc.at[dst_v.at[j]], add=True)

            @pl.when(j % 2 == 1)
            def _():
                pltpu.make_async_copy(g_hbm.at[src_v.at[j]], buf1, sem1).wait()

                @pl.when(nxt < KG)
                def _():
                    pltpu.make_async_copy(g_hbm.at[src_v.at[nxt]], buf0, sem0).start()

                pltpu.sync_copy(buf1, acc.at[dst_v.at[j]], add=True)

            return c2

        lax.fori_loop(0, KG, body, 0)
        return carry

    lax.fori_loop(0, G, group, 0)
    plsc.subcore_barrier()
    pltpu.sync_copy(acc.at[pl.ds(si * RPS, RPS)],
                    out_hbm.at[ci].at[pl.ds(si * RPS, RPS)])


def _tc0_body(x_ref, w_ref, degp_ref, g_ref, dinv_ref):
    deg = 1.0 + degp_ref[0, :N, 0:1] + degp_ref[1, :N, 0:1]
    dinv = lax.rsqrt(deg)
    dinv_ref[...] = dinv
    z = jnp.dot(x_ref[...], w_ref[...], preferred_element_type=jnp.float32)
    g_ref[...] = z * dinv


def _tc_mid_body(p_ref, g_ref, dinv_ref, b_ref, lnw_ref, lnb_ref, wn_ref,
                 h_ref, gn_ref):
    dinv = dinv_ref[...]
    u = (p_ref[0, :N] + p_ref[1, :N] + g_ref[...]) * dinv + b_ref[...]
    mean = jnp.mean(u)
    var = jnp.mean((u - mean) ** 2)
    t = (u - mean) * lax.rsqrt(var + EPS) * lnw_ref[...] + lnb_ref[...]
    h = jnp.maximum(t, 0.0)
    h_ref[...] = h
    z = jnp.dot(h, wn_ref[...], preferred_element_type=jnp.float32)
    gn_ref[...] = z * dinv


def _tc_last_body(p_ref, g_ref, dinv_ref, b_ref, lnw_ref, lnb_ref, h_ref):
    u = (p_ref[0, :N] + p_ref[1, :N] + g_ref[...]) * dinv_ref[...] + b_ref[...]
    mean = jnp.mean(u)
    var = jnp.mean((u - mean) ** 2)
    t = (u - mean) * lax.rsqrt(var + EPS) * lnw_ref[...] + lnb_ref[...]
    h_ref[...] = jnp.maximum(t, 0.0)


def _tc_head_body(h1_ref, h2_ref, h3_ref, mask_ref, wjk_ref, bjk_ref,
                  wfc1_ref, bfc1_ref, wfc2_ref, bfc2_ref, out_ref):
    f = (jnp.dot(h1_ref[...], wjk_ref[0:H], preferred_element_type=jnp.float32)
         + jnp.dot(h2_ref[...], wjk_ref[H:2 * H], preferred_element_type=jnp.float32)
         + jnp.dot(h3_ref[...], wjk_ref[2 * H:3 * H], preferred_element_type=jnp.float32)
         + bjk_ref[...])
    t = jnp.tanh(jnp.dot(f, wfc1_ref[...], preferred_element_type=jnp.float32)
                 + bfc1_ref[...])
    s = jnp.dot(t, wfc2_ref[...], preferred_element_type=jnp.float32) + bfc2_ref[...]
    s = s + mask_ref[...] * -1000000000.0
    m = jnp.max(s)
    e = jnp.exp(s - m)
    out_ref[...] = e / jnp.sum(e)


def _tc_call(body, out_shapes, *args):
    return pl.pallas_call(
        body,
        out_shape=out_shapes,
    )(*args)


def kernel(x, edge_index, mask, W1, b1, W2, b2, W3, b3, ln_w1, ln_b1,
           ln_w2, ln_b2, ln_w3, ln_b3, Wjk, bjk, Wfc1, bfc1, Wfc2, bfc2):
    src = edge_index[0].reshape(NW, G, KG, C)
    dst = edge_index[1].reshape(NW, G, KG, C)
    ones_d = jnp.ones((C, DW), jnp.float32)
    zeros_h = jnp.zeros((NP, H), jnp.float32)

    degp = _sc_degree(dst, ones_d)

    f32 = jnp.float32
    g1, dinv = _tc_call(
        _tc0_body,
        (jax.ShapeDtypeStruct((N, H), f32), jax.ShapeDtypeStruct((N, 1), f32)),
        x, W1, degp)

    nh = jax.ShapeDtypeStruct((N, H), f32)
    p1 = _sc_segsum(g1, src, dst, zeros_h)
    h1, g2 = _tc_call(
        _tc_mid_body, (nh, nh),
        p1, g1, dinv, b1.reshape(1, H), ln_w1.reshape(1, H),
        ln_b1.reshape(1, H), W2)

    p2 = _sc_segsum(g2, src, dst, zeros_h)
    h2, g3 = _tc_call(
        _tc_mid_body, (nh, nh),
        p2, g2, dinv, b2.reshape(1, H), ln_w2.reshape(1, H),
        ln_b2.reshape(1, H), W3)

    p3 = _sc_segsum(g3, src, dst, zeros_h)
    h3 = _tc_call(
        _tc_last_body, nh,
        p3, g3, dinv, b3.reshape(1, H), ln_w3.reshape(1, H),
        ln_b3.reshape(1, H))

    prob = _tc_call(
        _tc_head_body, jax.ShapeDtypeStruct((N, 1), f32),
        h1, h2, h3, mask.astype(f32), Wjk, bjk.reshape(1, H),
        Wfc1, bfc1.reshape(1, H // 2), Wfc2, bfc2.reshape(1, 1))

    return prob.reshape(N)


# speedup vs baseline: 3.0430x; 1.3438x over previous
import functools

import jax
import jax.numpy as jnp
from jax import lax
from jax.experimental import pallas as pl
from jax.experimental.pallas import tpu as pltpu
from jax.experimental.pallas import tpu_sc as plsc

N = 10000
E = 320000
H = 128
EPS = 1e-5

NC = 2
NS = 16
NW = NC * NS
NP = 10240
RPS = NP // NS
DW = 128
EPW = E // NW
C = 80
K = EPW // C
KG = 25
G = K // KG
ZT = RPS // C

_mesh = plsc.VectorSubcoreMesh(core_axis_name="c", subcore_axis_name="s")


def _zero_stripe(buf, acc, si):
    zv = jnp.zeros((16,), jnp.float32)
    width = buf.shape[1]

    def zrow(r, c2):
        for q in range(width // 16):
            buf[r, pl.ds(q * 16, 16)] = zv
        return c2

    lax.fori_loop(0, C, zrow, 0)
    for t in range(ZT):
        pltpu.sync_copy(buf, acc.at[pl.ds(si * RPS + t * C, C)])

@functools.partial(
    pl.kernel,
    out_type=jax.ShapeDtypeStruct((NC, NP, DW), jnp.float32),
    mesh=_mesh,
    scratch_types=[
        pltpu.VMEM((KG, C), jnp.int32),
        pltpu.VMEM((C, DW), jnp.float32),
        pltpu.VMEM_SHARED((NP, DW), jnp.float32),
    ],
)
def _sc_degree(dst_hbm, ones_hbm, out_hbm, dst_v, ones_v, acc):
    ci = lax.axis_index("c")
    si = lax.axis_index("s")
    wid = si * NC + ci
    _zero_stripe(ones_v, acc, si)
    pltpu.sync_copy(ones_hbm, ones_v)
    plsc.subcore_barrier()

    def group(gr, carry):
        pltpu.sync_copy(dst_hbm.at[wid].at[gr], dst_v)

        def body(j, c2):
            pltpu.sync_copy(ones_v, acc.at[dst_v.at[j]], add=True)
            return c2

        lax.fori_loop(0, KG, body, 0)
        return carry

    lax.fori_loop(0, G, group, 0)
    plsc.subcore_barrier()
    pltpu.sync_copy(acc.at[pl.ds(si * RPS, RPS)],
                    out_hbm.at[ci].at[pl.ds(si * RPS, RPS)])


@functools.partial(
    pl.kernel,
    out_type=jax.ShapeDtypeStruct((NC, NP, H), jnp.float32),
    mesh=_mesh,
    scratch_types=[
        pltpu.VMEM((KG, C), jnp.int32),
        pltpu.VMEM((KG, C), jnp.int32),
        pltpu.VMEM((C, H), jnp.float32),
        pltpu.VMEM((C, H), jnp.float32),
        pltpu.VMEM((C, H), jnp.float32),
        pltpu.VMEM_SHARED((NP, H), jnp.float32),
        pltpu.SemaphoreType.DMA,
        pltpu.SemaphoreType.DMA,
        pltpu.SemaphoreType.DMA,
    ],
)
def _sc_segsum(g_hbm, src_hbm, dst_hbm, out_hbm,
               src_v, dst_v, buf0, buf1, buf2, acc, sem0, sem1, sem2):
    ci = lax.axis_index("c")
    si = lax.axis_index("s")
    wid = si * NC + ci
    _zero_stripe(buf0, acc, si)
    plsc.subcore_barrier()

    def group(gr, carry):
        pltpu.sync_copy(src_hbm.at[wid].at[gr], src_v)
        pltpu.sync_copy(dst_hbm.at[wid].at[gr], dst_v)

        pltpu.make_async_copy(g_hbm.at[src_v.at[0]], buf0, sem0).start()
        pltpu.make_async_copy(g_hbm.at[src_v.at[1]], buf1, sem1).start()

        bufs = (buf0, buf1, buf2)
        sems = (sem0, sem1, sem2)

        def body(j, c2):
            for p in range(3):
                @pl.when(j % 3 == p)
                def _(p=p):
                    pltpu.make_async_copy(g_hbm.at[src_v.at[j]], bufs[p],
                                          sems[p]).wait()

                    @pl.when(j + 2 < KG)
                    def _():
                        q = (p + 2) % 3
                        pltpu.make_async_copy(g_hbm.at[src_v.at[j + 2]],
                                              bufs[q], sems[q]).start()

                    pltpu.sync_copy(bufs[p], acc.at[dst_v.at[j]], add=True)

            return c2

        lax.fori_loop(0, KG, body, 0)
        return carry

    lax.fori_loop(0, G, group, 0)
    plsc.subcore_barrier()
    pltpu.sync_copy(acc.at[pl.ds(si * RPS, RPS)],
                    out_hbm.at[ci].at[pl.ds(si * RPS, RPS)])


def _tc0_body(x_ref, w_ref, degp_ref, g_ref, dinv_ref):
    deg = 1.0 + degp_ref[0, :N, 0:1] + degp_ref[1, :N, 0:1]
    dinv = lax.rsqrt(deg)
    dinv_ref[...] = dinv
    z = jnp.dot(x_ref[...], w_ref[...], preferred_element_type=jnp.float32)
    g_ref[...] = z * dinv


def _tc_mid_body(p_ref, g_ref, dinv_ref, b_ref, lnw_ref, lnb_ref, wn_ref,
                 h_ref, gn_ref):
    dinv = dinv_ref[...]
    u = (p_ref[0, :N] + p_ref[1, :N] + g_ref[...]) * dinv + b_ref[...]
    mean = jnp.mean(u)
    var = jnp.mean((u - mean) ** 2)
    t = (u - mean) * lax.rsqrt(var + EPS) * lnw_ref[...] + lnb_ref[...]
    h = jnp.maximum(t, 0.0)
    h_ref[...] = h
    z = jnp.dot(h, wn_ref[...], preferred_element_type=jnp.float32)
    gn_ref[...] = z * dinv


def _tc_last_body(p_ref, g_ref, dinv_ref, b_ref, lnw_ref, lnb_ref, h_ref):
    u = (p_ref[0, :N] + p_ref[1, :N] + g_ref[...]) * dinv_ref[...] + b_ref[...]
    mean = jnp.mean(u)
    var = jnp.mean((u - mean) ** 2)
    t = (u - mean) * lax.rsqrt(var + EPS) * lnw_ref[...] + lnb_ref[...]
    h_ref[...] = jnp.maximum(t, 0.0)


def _tc_head_body(h1_ref, h2_ref, h3_ref, mask_ref, wjk_ref, bjk_ref,
                  wfc1_ref, bfc1_ref, wfc2_ref, bfc2_ref, out_ref):
    f = (jnp.dot(h1_ref[...], wjk_ref[0:H], preferred_element_type=jnp.float32)
         + jnp.dot(h2_ref[...], wjk_ref[H:2 * H], preferred_element_type=jnp.float32)
         + jnp.dot(h3_ref[...], wjk_ref[2 * H:3 * H], preferred_element_type=jnp.float32)
         + bjk_ref[...])
    t = jnp.tanh(jnp.dot(f, wfc1_ref[...], preferred_element_type=jnp.float32)
                 + bfc1_ref[...])
    s = jnp.dot(t, wfc2_ref[...], preferred_element_type=jnp.float32) + bfc2_ref[...]
    s = s + mask_ref[...] * -1000000000.0
    m = jnp.max(s)
    e = jnp.exp(s - m)
    out_ref[...] = e / jnp.sum(e)


def _tc_call(body, out_shapes, *args):
    return pl.pallas_call(
        body,
        out_shape=out_shapes,
    )(*args)


def kernel(x, edge_index, mask, W1, b1, W2, b2, W3, b3, ln_w1, ln_b1,
           ln_w2, ln_b2, ln_w3, ln_b3, Wjk, bjk, Wfc1, bfc1, Wfc2, bfc2):
    src = edge_index[0].reshape(NW, G, KG, C)
    dst = edge_index[1].reshape(NW, G, KG, C)
    ones_d = jnp.ones((C, DW), jnp.float32)

    degp = _sc_degree(dst, ones_d)

    f32 = jnp.float32
    g1, dinv = _tc_call(
        _tc0_body,
        (jax.ShapeDtypeStruct((N, H), f32), jax.ShapeDtypeStruct((N, 1), f32)),
        x, W1, degp)

    nh = jax.ShapeDtypeStruct((N, H), f32)
    p1 = _sc_segsum(g1, src, dst)
    h1, g2 = _tc_call(
        _tc_mid_body, (nh, nh),
        p1, g1, dinv, b1.reshape(1, H), ln_w1.reshape(1, H),
        ln_b1.reshape(1, H), W2)

    p2 = _sc_segsum(g2, src, dst)
    h2, g3 = _tc_call(
        _tc_mid_body, (nh, nh),
        p2, g2, dinv, b2.reshape(1, H), ln_w2.reshape(1, H),
        ln_b2.reshape(1, H), W3)

    p3 = _sc_segsum(g3, src, dst)
    h3 = _tc_call(
        _tc_last_body, nh,
        p3, g3, dinv, b3.reshape(1, H), ln_w3.reshape(1, H),
        ln_b3.reshape(1, H))

    prob = _tc_call(
        _tc_head_body, jax.ShapeDtypeStruct((N, 1), f32),
        h1, h2, h3, mask.astype(f32), Wjk, bjk.reshape(1, H),
        Wfc1, bfc1.reshape(1, H // 2), Wfc2, bfc2.reshape(1, 1))

    return prob.reshape(N)
